# baseline (speedup 1.0000x reference)
import jax
import jax.numpy as jnp
from jax.experimental import pallas as pl

_D = 768
_T = 32
_K = 5
_GPB = 64
_RB = _GPB * _T
_NEG = -1e9


def _fused(wo_ref, x_ref, w1_ref, b1_ref, w2_ref, b2_ref,
           filt_ref, sc_ref, attn_ref, keep_ref, idx_ref, tv_ref):
    wo = wo_ref[...]
    filt_ref[...] = wo
    h = jnp.tanh(
        jax.lax.dot_general(
            wo, w1_ref[...], (((1,), (1,)), ((), ())),
            preferred_element_type=jnp.float32) + b1_ref[...])
    w2rep = jnp.broadcast_to(w2_ref[...], (_T, _D))
    sb = jax.lax.dot_general(
        h, w2rep, (((1,), (1,)), ((), ())),
        preferred_element_type=jnp.float32)
    rr = jax.lax.broadcasted_iota(jnp.int32, (_RB, _T), 0)
    ll = jax.lax.broadcasted_iota(jnp.int32, (_RB, _T), 1)
    sd = jnp.where((rr % _T) == ll, sb, 0.0)
    s2 = jnp.sum(sd.reshape(_GPB, _T, _T), axis=1) + b2_ref[0, 0]
    pad = x_ref[...] == 0
    sm = jnp.where(pad, _NEG, s2)
    sc_ref[...] = sm
    mx = jnp.max(sm, axis=1, keepdims=True)
    ex = jnp.exp(sm - mx)
    attn_ref[...] = ex / jnp.sum(ex, axis=1, keepdims=True)

    it = jax.lax.broadcasted_iota(jnp.int32, (_GPB, _T), 1)
    work = sm
    keep = jnp.zeros((_GPB, _T), jnp.float32)
    cols = []
    for _ in range(_K):
        mj = jnp.max(work, axis=1, keepdims=True)
        aj = jnp.min(jnp.where(work == mj, it, _T), axis=1, keepdims=True)
        hit = it == aj
        keep = jnp.where(hit, 1.0, keep)
        work = jnp.where(hit, -jnp.inf, work)
        cols.append(aj)
    idx_ref[...] = jnp.concatenate(cols, axis=1)
    keep_ref[...] = jnp.where(pad, 0.0, keep)

    gg2 = jax.lax.broadcasted_iota(jnp.int32, (_GPB, _RB), 0)
    rr2 = jax.lax.broadcasted_iota(jnp.int32, (_GPB, _RB), 1)
    grp_ok = (rr2 // _T) == gg2
    tmod = rr2 % _T
    wo_b = wo.astype(jnp.bfloat16)
    for j in range(_K):
        pj = (jnp.broadcast_to(cols[j], (_GPB, _RB)) == tmod) & grp_ok
        tv_ref[:, j, :] = jax.lax.dot_general(
            pj.astype(jnp.bfloat16), wo_b, (((1,), (0,)), ((), ())),
            preferred_element_type=jnp.float32)


def kernel(word_out, x, W1, b1, W2, b2):
    B, S, T, D = word_out.shape
    BS = B * S
    nblk = (BS * T) // _RB
    wo2 = word_out.reshape(BS * T, D)
    x2 = x.reshape(BS, T).astype(jnp.int32)
    b1r = b1.reshape(1, D)
    b2r = b2.reshape(1, 1)

    filt, sc, attn, keep, idx, tv = pl.pallas_call(
        _fused,
        grid=(nblk,),
        in_specs=[
            pl.BlockSpec((_RB, D), lambda i: (i, 0)),
            pl.BlockSpec((_GPB, _T), lambda i: (i, 0)),
            pl.BlockSpec((D, D), lambda i: (0, 0)),
            pl.BlockSpec((1, D), lambda i: (0, 0)),
            pl.BlockSpec((1, D), lambda i: (0, 0)),
            pl.BlockSpec((1, 1), lambda i: (0, 0)),
        ],
        out_specs=[
            pl.BlockSpec((_RB, D), lambda i: (i, 0)),
            pl.BlockSpec((_GPB, _T), lambda i: (i, 0)),
            pl.BlockSpec((_GPB, _T), lambda i: (i, 0)),
            pl.BlockSpec((_GPB, _T), lambda i: (i, 0)),
            pl.BlockSpec((_GPB, _K), lambda i: (i, 0)),
            pl.BlockSpec((_GPB, _K, D), lambda i: (i, 0, 0)),
        ],
        out_shape=[
            jax.ShapeDtypeStruct((BS * T, D), jnp.float32),
            jax.ShapeDtypeStruct((BS, T), jnp.float32),
            jax.ShapeDtypeStruct((BS, T), jnp.float32),
            jax.ShapeDtypeStruct((BS, T), jnp.float32),
            jax.ShapeDtypeStruct((BS, _K), jnp.int32),
            jax.ShapeDtypeStruct((BS, _K, D), jnp.float32),
        ],
    )(wo2, x2, W1, b1r, W2, b2r)

    filtered = filt.reshape(B, S, T, D)
    top_word_vecs = tv.reshape(B, S, _K, D)
    return (filtered, sc.reshape(B, S, T), keep.reshape(B, S, T),
            attn.reshape(B, S, T), idx.reshape(B, S, _K), top_word_vecs)


# speedup vs baseline: 1.0969x; 1.0969x over previous
import jax
import jax.numpy as jnp
from jax.experimental import pallas as pl

_D = 768
_T = 32
_K = 5
_GPB = 64
_RB = _GPB * _T
_NEG = -1e9


def _fused(wo_ref, x_ref, w1_ref, b1_ref, w2_ref, b2_ref,
           filt_ref, sc_ref, attn_ref, keep_ref, idx_ref, tv_ref):
    wo = wo_ref[...].reshape(_RB, _D)
    filt_ref[...] = wo_ref[...]
    h = jnp.tanh(
        jax.lax.dot_general(
            wo, w1_ref[...], (((1,), (1,)), ((), ())),
            preferred_element_type=jnp.float32) + b1_ref[...])
    w2rep = jnp.broadcast_to(w2_ref[...], (_T, _D))
    sb = jax.lax.dot_general(
        h, w2rep, (((1,), (1,)), ((), ())),
        preferred_element_type=jnp.float32)
    rr = jax.lax.broadcasted_iota(jnp.int32, (_RB, _T), 0)
    ll = jax.lax.broadcasted_iota(jnp.int32, (_RB, _T), 1)
    sd = jnp.where((rr % _T) == ll, sb, 0.0)
    s2 = jnp.sum(sd.reshape(_GPB, _T, _T), axis=1) + b2_ref[0, 0]
    pad = x_ref[...].reshape(_GPB, _T) == 0
    sm = jnp.where(pad, _NEG, s2)
    sc_ref[...] = sm.reshape(1, _GPB, _T)
    mx = jnp.max(sm, axis=1, keepdims=True)
    ex = jnp.exp(sm - mx)
    attn_ref[...] = (ex / jnp.sum(ex, axis=1, keepdims=True)).reshape(1, _GPB, _T)

    it = jax.lax.broadcasted_iota(jnp.int32, (_GPB, _T), 1)
    work = sm
    keep = jnp.zeros((_GPB, _T), jnp.float32)
    cols = []
    for _ in range(_K):
        mj = jnp.max(work, axis=1, keepdims=True)
        aj = jnp.min(jnp.where(work == mj, it, _T), axis=1, keepdims=True)
        hit = it == aj
        keep = jnp.where(hit, 1.0, keep)
        work = jnp.where(hit, -jnp.inf, work)
        cols.append(aj)
    idx_ref[...] = jnp.concatenate(cols, axis=1).reshape(1, _GPB, _K)
    keep_ref[...] = jnp.where(pad, 0.0, keep).reshape(1, _GPB, _T)

    gg2 = jax.lax.broadcasted_iota(jnp.int32, (_GPB, _RB), 0)
    rr2 = jax.lax.broadcasted_iota(jnp.int32, (_GPB, _RB), 1)
    grp_ok = (rr2 // _T) == gg2
    tmod = rr2 % _T
    wo_b = wo.astype(jnp.bfloat16)
    for j in range(_K):
        pj = (jnp.broadcast_to(cols[j], (_GPB, _RB)) == tmod) & grp_ok
        tv_ref[0, :, j, :] = jax.lax.dot_general(
            pj.astype(jnp.bfloat16), wo_b, (((1,), (0,)), ((), ())),
            preferred_element_type=jnp.float32)


def kernel(word_out, x, W1, b1, W2, b2):
    B, S, T, D = word_out.shape
    x3 = x.astype(jnp.int32)
    b1r = b1.reshape(1, D)
    b2r = b2.reshape(1, 1)

    filt, sc, attn, keep, idx, tv = pl.pallas_call(
        _fused,
        grid=(B,),
        in_specs=[
            pl.BlockSpec((1, S, T, D), lambda i: (i, 0, 0, 0)),
            pl.BlockSpec((1, S, T), lambda i: (i, 0, 0)),
            pl.BlockSpec((D, D), lambda i: (0, 0)),
            pl.BlockSpec((1, D), lambda i: (0, 0)),
            pl.BlockSpec((1, D), lambda i: (0, 0)),
            pl.BlockSpec((1, 1), lambda i: (0, 0)),
        ],
        out_specs=[
            pl.BlockSpec((1, S, T, D), lambda i: (i, 0, 0, 0)),
            pl.BlockSpec((1, S, T), lambda i: (i, 0, 0)),
            pl.BlockSpec((1, S, T), lambda i: (i, 0, 0)),
            pl.BlockSpec((1, S, T), lambda i: (i, 0, 0)),
            pl.BlockSpec((1, S, _K), lambda i: (i, 0, 0)),
            pl.BlockSpec((1, S, _K, D), lambda i: (i, 0, 0, 0)),
        ],
        out_shape=[
            jax.ShapeDtypeStruct((B, S, T, D), jnp.float32),
            jax.ShapeDtypeStruct((B, S, T), jnp.float32),
            jax.ShapeDtypeStruct((B, S, T), jnp.float32),
            jax.ShapeDtypeStruct((B, S, T), jnp.float32),
            jax.ShapeDtypeStruct((B, S, _K), jnp.int32),
            jax.ShapeDtypeStruct((B, S, _K, D), jnp.float32),
        ],
    )(word_out, x3, W1, b1r, W2, b2r)

    return (filt, sc, keep, attn, idx, tv)
